# Initial kernel scaffold; baseline (speedup 1.0000x reference)
import functools

import jax
import jax.numpy as jnp
from jax import lax
from jax.experimental import pallas as pl
from jax.experimental.pallas import tpu as pltpu
from jax.experimental.pallas import tpu_sc as plsc

N = 10000
E = 320000
D_IN = 512
C = 64
G = 16
P = 50000

NC = 2
NS = 16
NW = NC * NS

ECH = 125
EW = E // NW
NCH = EW // ECH

PPAD = 52000
PCH = PPAD // (NW * ECH)
PW = PPAD // NW

ROWS_S = N // NS

_mesh = plsc.VectorSubcoreMesh(core_axis_name="c", subcore_axis_name="s",
                               num_cores=NC, num_subcores=NS)


@functools.partial(
    pl.kernel,
    out_type=jax.ShapeDtypeStruct((NC, N, C), jnp.float32),
    mesh=_mesh,
    scratch_types=[
        pltpu.VMEM((NCH, ECH), jnp.int32),
        pltpu.VMEM((NCH, ECH), jnp.int32),
        pltpu.VMEM((ECH, C), jnp.float32),
        pltpu.VMEM_SHARED((N, C), jnp.float32),
        pltpu.SemaphoreType.DMA,
    ],
)
def _sc_agg(h_hbm, src_hbm, dst_hbm, zeros_hbm, out_hbm,
            src_v, dst_v, rows_v, acc, sem):
    c = lax.axis_index("c")
    s = lax.axis_index("s")
    wid = c * NS + s
    pltpu.sync_copy(src_hbm.at[wid], src_v)
    pltpu.sync_copy(dst_hbm.at[wid], dst_v)
    pltpu.sync_copy(zeros_hbm.at[pl.ds(s * ROWS_S, ROWS_S)],
                    acc.at[pl.ds(s * ROWS_S, ROWS_S)])
    plsc.subcore_barrier()

    def body(j, carry):
        pltpu.async_copy(h_hbm.at[src_v.at[j]], rows_v, sem).wait()
        pltpu.sync_copy(rows_v, acc.at[dst_v.at[j]], add=True)
        return carry

    lax.fori_loop(0, NCH, body, 0)
    plsc.subcore_barrier()
    pltpu.sync_copy(acc.at[pl.ds(s * ROWS_S, ROWS_S)],
                    out_hbm.at[c, pl.ds(s * ROWS_S, ROWS_S)])


@functools.partial(
    pl.kernel,
    out_type=jax.ShapeDtypeStruct((NC, N, 1), jnp.float32),
    mesh=_mesh,
    scratch_types=[
        pltpu.VMEM((NCH, ECH), jnp.int32),
        pltpu.VMEM((ECH, 1), jnp.float32),
        pltpu.VMEM_SHARED((N, 1), jnp.float32),
    ],
)
def _sc_cnt(dst_hbm, ones_hbm, zeros_hbm, out_hbm, dst_v, ones_v, acc):
    c = lax.axis_index("c")
    s = lax.axis_index("s")
    wid = c * NS + s
    pltpu.sync_copy(dst_hbm.at[wid], dst_v)
    pltpu.sync_copy(ones_hbm, ones_v)
    pltpu.sync_copy(zeros_hbm.at[pl.ds(s * ROWS_S, ROWS_S)],
                    acc.at[pl.ds(s * ROWS_S, ROWS_S)])
    plsc.subcore_barrier()

    def body(j, carry):
        pltpu.sync_copy(ones_v, acc.at[dst_v.at[j]], add=True)
        return carry

    lax.fori_loop(0, NCH, body, 0)
    plsc.subcore_barrier()
    pltpu.sync_copy(acc.at[pl.ds(s * ROWS_S, ROWS_S)],
                    out_hbm.at[c, pl.ds(s * ROWS_S, ROWS_S)])


@functools.partial(
    pl.kernel,
    out_type=(jax.ShapeDtypeStruct((PPAD, 2 * C), jnp.float32),
              jax.ShapeDtypeStruct((PPAD, C), jnp.float32)),
    mesh=_mesh,
    scratch_types=[
        pltpu.VMEM((PCH, ECH), jnp.int32),
        pltpu.VMEM((PCH, ECH), jnp.int32),
        pltpu.VMEM((ECH, 2 * C), jnp.float32),
        pltpu.VMEM((ECH, C), jnp.float32),
        pltpu.SemaphoreType.DMA,
    ],
)
def _sc_pairs(hg_hbm, h_hbm, si_hbm, di_hbm, sg_out, d_out,
              si_v, di_v, sgrow, drow, sem):
    c = lax.axis_index("c")
    s = lax.axis_index("s")
    wid = c * NS + s
    base = wid * PW
    pltpu.sync_copy(si_hbm.at[wid], si_v)
    pltpu.sync_copy(di_hbm.at[wid], di_v)

    def body(j, carry):
        pltpu.async_copy(hg_hbm.at[si_v.at[j]], sgrow, sem).wait()
        pltpu.sync_copy(sgrow, sg_out.at[pl.ds(base + j * ECH, ECH)])
        pltpu.async_copy(h_hbm.at[di_v.at[j]], drow, sem).wait()
        pltpu.sync_copy(drow, d_out.at[pl.ds(base + j * ECH, ECH)])
        return carry

    lax.fori_loop(0, PCH, body, 0)


BR_E = 2000
BR_L = 2000
BR_G = 1000
BR_H = 2000
BR_P = 2500


def _emb_body(x_ref, m_ref, t_ref, we_ref, wt_ref, be_ref, o_ref):
    xm = x_ref[...] * m_ref[...]
    y = jnp.dot(xm, we_ref[...], preferred_element_type=jnp.float32)
    y = y + t_ref[...] * wt_ref[...] + be_ref[...]
    o_ref[...] = jax.nn.gelu(y)


def _sage_body(h_ref, agg_ref, cnt_ref, wl_ref, wr_ref, bl_ref, o_ref):
    h = h_ref[...]
    aggs = agg_ref[0] + agg_ref[1]
    cnts = cnt_ref[0] + cnt_ref[1]
    mean = aggs / jnp.maximum(cnts, 1.0)
    y = (jnp.dot(mean, wl_ref[...], preferred_element_type=jnp.float32)
         + jnp.dot(h, wr_ref[...], preferred_element_type=jnp.float32)
         + bl_ref[...])
    o_ref[...] = h + jax.nn.gelu(y)


def _gmax_body(h_ref, b_ref, o_ref):
    i = pl.program_id(0)

    @pl.when(i == 0)
    def _():
        o_ref[...] = jnp.full((G, C), -jnp.inf, jnp.float32)

    h = h_ref[...]
    b = b_ref[...]
    rows = [jnp.where(b == g, h, -jnp.inf).max(axis=0) for g in range(G)]
    o_ref[...] = jnp.maximum(o_ref[...], jnp.stack(rows))


def _hg_body(h_ref, b_ref, ge_ref, o_ref):
    o_ref[:, 0:C] = h_ref[...]
    b = b_ref[...]
    oh = (b == lax.broadcasted_iota(jnp.float32, (BR_H, G), 1))
    o_ref[:, C:2 * C] = jnp.dot(oh.astype(jnp.float32), ge_ref[...],
                                preferred_element_type=jnp.float32)


def _head_body(sg_ref, d_ref, w1_ref, b1_ref, w2_ref, b2_ref,
               w3_ref, b3_ref, w4_ref, b4_ref, xy_ref, pxy_ref, pyx_ref):
    sg = sg_ref[...]
    s = sg[:, 0:C]
    g = sg[:, C:2 * C]
    d = d_ref[...]
    xy_ref[:, 0:C] = s
    xy_ref[:, C:2 * C] = d
    xy_ref[:, 2 * C:3 * C] = g
    w1 = w1_ref[...]
    wa = w1[0:C]
    wb = w1[C:2 * C]
    wg = w1[2 * C:3 * C]
    f32 = jnp.float32
    gg = jnp.dot(g, wg, preferred_element_type=f32) + b1_ref[...]
    sa = jnp.dot(s, wa, preferred_element_type=f32)
    sb = jnp.dot(s, wb, preferred_element_type=f32)
    da = jnp.dot(d, wa, preferred_element_type=f32)
    db = jnp.dot(d, wb, preferred_element_type=f32)
    h1xy = jax.nn.relu(sa + db + gg)
    h1yx = jax.nn.relu(da + sb + gg)

    def tail(h1):
        h2 = jax.nn.relu(jnp.dot(h1, w2_ref[...], preferred_element_type=f32)
                         + b2_ref[...])
        h3 = jax.nn.relu(jnp.dot(h2, w3_ref[...], preferred_element_type=f32)
                         + b3_ref[...])
        return jnp.dot(h3, w4_ref[...], preferred_element_type=f32) + b4_ref[...]

    pxy_ref[...] = tail(h1xy)
    pyx_ref[...] = tail(h1yx)


def _full(shape):
    return pl.BlockSpec(shape, lambda i: tuple(0 for _ in shape))


def kernel(x, mask, times, edge_index, batch_ids, src_idx, dst_idx,
           W_emb, b_emb, W_time, b_time,
           Wl1, bl1, Wr1, Wl2, bl2, Wr2, Wl3, bl3, Wr3,
           W1, b1, W2, b2, W3, b3, W4, b4):
    f32 = jnp.float32
    i32 = jnp.int32

    src = edge_index[0].astype(i32).reshape(NW, NCH, ECH)
    dst = edge_index[1].astype(i32).reshape(NW, NCH, ECH)
    si = jnp.concatenate([src_idx.astype(i32),
                          jnp.zeros((PPAD - P,), i32)]).reshape(NW, PCH, ECH)
    di = jnp.concatenate([dst_idx.astype(i32),
                          jnp.zeros((PPAD - P,), i32)]).reshape(NW, PCH, ECH)
    batch_f = batch_ids.astype(f32).reshape(N, 1)

    zeros64 = jnp.zeros((N, C), f32)
    zeros1 = jnp.zeros((N, 1), f32)
    ones1 = jnp.ones((ECH, 1), f32)

    be = (b_emb + b_time).reshape(1, C)
    wt = W_time.reshape(1, C)

    h = pl.pallas_call(
        _emb_body,
        grid=(N // BR_E,),
        in_specs=[
            pl.BlockSpec((BR_E, D_IN), lambda i: (i, 0)),
            pl.BlockSpec((BR_E, D_IN), lambda i: (i, 0)),
            pl.BlockSpec((BR_E, 1), lambda i: (i, 0)),
            _full((D_IN, C)),
            _full((1, C)),
            _full((1, C)),
        ],
        out_specs=pl.BlockSpec((BR_E, C), lambda i: (i, 0)),
        out_shape=jax.ShapeDtypeStruct((N, C), f32),
    )(x, mask, times, W_emb, wt, be)

    cnt2 = _sc_cnt(dst, ones1, zeros1)

    for wl, bl, wr in ((Wl1, bl1, Wr1), (Wl2, bl2, Wr2), (Wl3, bl3, Wr3)):
        agg2 = _sc_agg(h, src, dst, zeros64)
        h = pl.pallas_call(
            _sage_body,
            grid=(N // BR_L,),
            in_specs=[
                pl.BlockSpec((BR_L, C), lambda i: (i, 0)),
                pl.BlockSpec((NC, BR_L, C), lambda i: (0, i, 0)),
                pl.BlockSpec((NC, BR_L, 1), lambda i: (0, i, 0)),
                _full((C, C)),
                _full((C, C)),
                _full((1, C)),
            ],
            out_specs=pl.BlockSpec((BR_L, C), lambda i: (i, 0)),
            out_shape=jax.ShapeDtypeStruct((N, C), f32),
        )(h, agg2, cnt2, wl, wr, bl.reshape(1, C))

    ge = pl.pallas_call(
        _gmax_body,
        grid=(N // BR_G,),
        in_specs=[
            pl.BlockSpec((BR_G, C), lambda i: (i, 0)),
            pl.BlockSpec((BR_G, 1), lambda i: (i, 0)),
        ],
        out_specs=_full((G, C)),
        out_shape=jax.ShapeDtypeStruct((G, C), f32),
    )(h, batch_f)

    hg = pl.pallas_call(
        _hg_body,
        grid=(N // BR_H,),
        in_specs=[
            pl.BlockSpec((BR_H, C), lambda i: (i, 0)),
            pl.BlockSpec((BR_H, 1), lambda i: (i, 0)),
            _full((G, C)),
        ],
        out_specs=pl.BlockSpec((BR_H, 2 * C), lambda i: (i, 0)),
        out_shape=jax.ShapeDtypeStruct((N, 2 * C), f32),
    )(h, batch_f, ge)

    sg, dg = _sc_pairs(hg, h, si, di)

    xy, pxy, pyx = pl.pallas_call(
        _head_body,
        grid=(P // BR_P,),
        in_specs=[
            pl.BlockSpec((BR_P, 2 * C), lambda i: (i, 0)),
            pl.BlockSpec((BR_P, C), lambda i: (i, 0)),
            _full((3 * C, 2 * C)),
            _full((1, 2 * C)),
            _full((2 * C, 2 * C)),
            _full((1, 2 * C)),
            _full((2 * C, 2 * C)),
            _full((1, 2 * C)),
            _full((2 * C, 1)),
            _full((1, 1)),
        ],
        out_specs=[
            pl.BlockSpec((BR_P, 3 * C), lambda i: (i, 0)),
            pl.BlockSpec((BR_P, 1), lambda i: (i, 0)),
            pl.BlockSpec((BR_P, 1), lambda i: (i, 0)),
        ],
        out_shape=[
            jax.ShapeDtypeStruct((P, 3 * C), f32),
            jax.ShapeDtypeStruct((P, 1), f32),
            jax.ShapeDtypeStruct((P, 1), f32),
        ],
    )(sg, dg, W1, b1.reshape(1, 2 * C), W2, b2.reshape(1, 2 * C),
      W3, b3.reshape(1, 2 * C), W4, b4.reshape(1, 1))

    return (h, (pxy, pyx), xy)


# speedup vs baseline: 5.8000x; 5.8000x over previous
import functools

import jax
import jax.numpy as jnp
from jax import lax
from jax.experimental import pallas as pl
from jax.experimental.pallas import tpu as pltpu
from jax.experimental.pallas import tpu_sc as plsc

N = 10000
E = 320000
D_IN = 512
C = 64
G = 16
P = 50000

NC = 2
NS = 16
NW = NC * NS

ECH = 125
EW = E // NW
NCH = EW // ECH

PPAD = 52000
PCH = PPAD // (NW * ECH)
PW = PPAD // NW

ROWS_S = N // NS

_mesh = plsc.VectorSubcoreMesh(core_axis_name="c", subcore_axis_name="s",
                               num_cores=NC, num_subcores=NS)


@functools.partial(
    pl.kernel,
    out_type=jax.ShapeDtypeStruct((NC, N, C), jnp.float32),
    mesh=_mesh,
    compiler_params=pltpu.CompilerParams(use_tc_tiling_on_sc=False),
    scratch_types=[
        pltpu.VMEM((NCH, ECH), jnp.int32),
        pltpu.VMEM((NCH, ECH), jnp.int32),
        pltpu.VMEM((ECH, C), jnp.float32),
        pltpu.VMEM_SHARED((N, C), jnp.float32),
        pltpu.SemaphoreType.DMA,
    ],
)
def _sc_agg(h_hbm, src_hbm, dst_hbm, zeros_hbm, out_hbm,
            src_v, dst_v, rows_v, acc, sem):
    c = lax.axis_index("c")
    s = lax.axis_index("s")
    wid = c * NS + s
    pltpu.sync_copy(src_hbm.at[wid], src_v)
    pltpu.sync_copy(dst_hbm.at[wid], dst_v)
    pltpu.sync_copy(zeros_hbm.at[pl.ds(s * ROWS_S, ROWS_S)],
                    acc.at[pl.ds(s * ROWS_S, ROWS_S)])
    plsc.subcore_barrier()

    def body(j, carry):
        pltpu.async_copy(h_hbm.at[src_v.at[j]], rows_v, sem).wait()
        pltpu.sync_copy(rows_v, acc.at[dst_v.at[j]], add=True)
        return carry

    lax.fori_loop(0, NCH, body, 0)
    plsc.subcore_barrier()
    pltpu.sync_copy(acc.at[pl.ds(s * ROWS_S, ROWS_S)],
                    out_hbm.at[c, pl.ds(s * ROWS_S, ROWS_S)])


CW = 8


@functools.partial(
    pl.kernel,
    out_type=jax.ShapeDtypeStruct((NC, N, CW), jnp.float32),
    mesh=_mesh,
    compiler_params=pltpu.CompilerParams(use_tc_tiling_on_sc=False),
    scratch_types=[
        pltpu.VMEM((NCH, ECH), jnp.int32),
        pltpu.VMEM((ECH, CW), jnp.float32),
        pltpu.VMEM_SHARED((N, CW), jnp.float32),
    ],
)
def _sc_cnt(dst_hbm, ones_hbm, zeros_hbm, out_hbm, dst_v, ones_v, acc):
    c = lax.axis_index("c")
    s = lax.axis_index("s")
    wid = c * NS + s
    pltpu.sync_copy(dst_hbm.at[wid], dst_v)
    pltpu.sync_copy(ones_hbm, ones_v)
    pltpu.sync_copy(zeros_hbm.at[pl.ds(s * ROWS_S, ROWS_S)],
                    acc.at[pl.ds(s * ROWS_S, ROWS_S)])
    plsc.subcore_barrier()

    def body(j, carry):
        pltpu.sync_copy(ones_v, acc.at[dst_v.at[j]], add=True)
        return carry

    lax.fori_loop(0, NCH, body, 0)
    plsc.subcore_barrier()
    pltpu.sync_copy(acc.at[pl.ds(s * ROWS_S, ROWS_S)],
                    out_hbm.at[c, pl.ds(s * ROWS_S, ROWS_S)])


@functools.partial(
    pl.kernel,
    out_type=(jax.ShapeDtypeStruct((PPAD, 2 * C), jnp.float32),
              jax.ShapeDtypeStruct((PPAD, C), jnp.float32)),
    mesh=_mesh,
    compiler_params=pltpu.CompilerParams(use_tc_tiling_on_sc=False),
    scratch_types=[
        pltpu.VMEM((PCH, ECH), jnp.int32),
        pltpu.VMEM((PCH, ECH), jnp.int32),
        pltpu.VMEM((ECH, 2 * C), jnp.float32),
        pltpu.VMEM((ECH, C), jnp.float32),
        pltpu.SemaphoreType.DMA,
    ],
)
def _sc_pairs(hg_hbm, h_hbm, si_hbm, di_hbm, sg_out, d_out,
              si_v, di_v, sgrow, drow, sem):
    c = lax.axis_index("c")
    s = lax.axis_index("s")
    wid = c * NS + s
    base = wid * PW
    pltpu.sync_copy(si_hbm.at[wid], si_v)
    pltpu.sync_copy(di_hbm.at[wid], di_v)

    def body(j, carry):
        pltpu.async_copy(hg_hbm.at[si_v.at[j]], sgrow, sem).wait()
        pltpu.sync_copy(sgrow, sg_out.at[pl.ds(base + j * ECH, ECH)])
        pltpu.async_copy(h_hbm.at[di_v.at[j]], drow, sem).wait()
        pltpu.sync_copy(drow, d_out.at[pl.ds(base + j * ECH, ECH)])
        return carry

    lax.fori_loop(0, PCH, body, 0)


BR_E = 2000
BR_L = 2000
BR_G = 1000
BR_H = 2000
BR_P = 2000


def _emb_body(x_ref, m_ref, t_ref, we_ref, wt_ref, be_ref, o_ref):
    xm = x_ref[...] * m_ref[...]
    y = jnp.dot(xm, we_ref[...], preferred_element_type=jnp.float32)
    y = y + t_ref[...] * wt_ref[...] + be_ref[...]
    o_ref[...] = jax.nn.gelu(y)


def _sage_body(h_ref, agg_ref, cnt_ref, wl_ref, wr_ref, bl_ref, o_ref):
    h = h_ref[...]
    aggs = agg_ref[0] + agg_ref[1]
    cnts = cnt_ref[0, :, 0:1] + cnt_ref[1, :, 0:1]
    mean = aggs / jnp.maximum(cnts, 1.0)
    y = (jnp.dot(mean, wl_ref[...], preferred_element_type=jnp.float32)
         + jnp.dot(h, wr_ref[...], preferred_element_type=jnp.float32)
         + bl_ref[...])
    o_ref[...] = h + jax.nn.gelu(y)


def _gmax_body(h_ref, b_ref, o_ref):
    i = pl.program_id(0)

    @pl.when(i == 0)
    def _():
        o_ref[...] = jnp.full((G, C), -jnp.inf, jnp.float32)

    h = h_ref[...]
    b = b_ref[...]
    rows = [jnp.where(b == g, h, -jnp.inf).max(axis=0) for g in range(G)]
    o_ref[...] = jnp.maximum(o_ref[...], jnp.stack(rows))


def _hg_body(h_ref, b_ref, ge_ref, o_ref):
    o_ref[:, 0:C] = h_ref[...]
    b = b_ref[...]
    oh = (b.astype(jnp.int32) == lax.broadcasted_iota(jnp.int32, (BR_H, G), 1))
    o_ref[:, C:2 * C] = jnp.dot(oh.astype(jnp.float32), ge_ref[...],
                                preferred_element_type=jnp.float32)


def _head_body(sg_ref, d_ref, w1_ref, b1_ref, w2_ref, b2_ref,
               w3_ref, b3_ref, w4_ref, b4_ref, xy_ref, pxy_ref, pyx_ref):
    sg = sg_ref[...]
    s = sg[:, 0:C]
    g = sg[:, C:2 * C]
    d = d_ref[...]
    xy_ref[:, 0:C] = s
    xy_ref[:, C:2 * C] = d
    xy_ref[:, 2 * C:3 * C] = g
    w1 = w1_ref[...]
    wa = w1[0:C]
    wb = w1[C:2 * C]
    wg = w1[2 * C:3 * C]
    f32 = jnp.float32
    gg = jnp.dot(g, wg, preferred_element_type=f32) + b1_ref[...]
    sa = jnp.dot(s, wa, preferred_element_type=f32)
    sb = jnp.dot(s, wb, preferred_element_type=f32)
    da = jnp.dot(d, wa, preferred_element_type=f32)
    db = jnp.dot(d, wb, preferred_element_type=f32)
    h1xy = jax.nn.relu(sa + db + gg)
    h1yx = jax.nn.relu(da + sb + gg)

    def tail(h1):
        h2 = jax.nn.relu(jnp.dot(h1, w2_ref[...], preferred_element_type=f32)
                         + b2_ref[...])
        h3 = jax.nn.relu(jnp.dot(h2, w3_ref[...], preferred_element_type=f32)
                         + b3_ref[...])
        return jnp.dot(h3, w4_ref[...], preferred_element_type=f32) + b4_ref[...]

    pxy_ref[...] = tail(h1xy)
    pyx_ref[...] = tail(h1yx)


def _full(shape):
    return pl.BlockSpec(shape, lambda i: tuple(0 for _ in shape))


def kernel(x, mask, times, edge_index, batch_ids, src_idx, dst_idx,
           W_emb, b_emb, W_time, b_time,
           Wl1, bl1, Wr1, Wl2, bl2, Wr2, Wl3, bl3, Wr3,
           W1, b1, W2, b2, W3, b3, W4, b4):
    f32 = jnp.float32
    i32 = jnp.int32

    src = edge_index[0].astype(i32).reshape(NW, NCH, ECH)
    dst = edge_index[1].astype(i32).reshape(NW, NCH, ECH)
    si = jnp.concatenate([src_idx.astype(i32),
                          jnp.zeros((PPAD - P,), i32)]).reshape(NW, PCH, ECH)
    di = jnp.concatenate([dst_idx.astype(i32),
                          jnp.zeros((PPAD - P,), i32)]).reshape(NW, PCH, ECH)
    batch_f = batch_ids.astype(f32).reshape(N, 1)

    zeros64 = jnp.zeros((N, C), f32)
    zeros1 = jnp.zeros((N, CW), f32)
    ones1 = jnp.ones((ECH, CW), f32)

    be = (b_emb + b_time).reshape(1, C)
    wt = W_time.reshape(1, C)

    h = pl.pallas_call(
        _emb_body,
        grid=(N // BR_E,),
        in_specs=[
            pl.BlockSpec((BR_E, D_IN), lambda i: (i, 0)),
            pl.BlockSpec((BR_E, D_IN), lambda i: (i, 0)),
            pl.BlockSpec((BR_E, 1), lambda i: (i, 0)),
            _full((D_IN, C)),
            _full((1, C)),
            _full((1, C)),
        ],
        out_specs=pl.BlockSpec((BR_E, C), lambda i: (i, 0)),
        out_shape=jax.ShapeDtypeStruct((N, C), f32),
    )(x, mask, times, W_emb, wt, be)

    cnt2 = _sc_cnt(dst, ones1, zeros1)

    for wl, bl, wr in ((Wl1, bl1, Wr1), (Wl2, bl2, Wr2), (Wl3, bl3, Wr3)):
        agg2 = _sc_agg(h, src, dst, zeros64)
        h = pl.pallas_call(
            _sage_body,
            grid=(N // BR_L,),
            in_specs=[
                pl.BlockSpec((BR_L, C), lambda i: (i, 0)),
                pl.BlockSpec((NC, BR_L, C), lambda i: (0, i, 0)),
                pl.BlockSpec((NC, BR_L, CW), lambda i: (0, i, 0)),
                _full((C, C)),
                _full((C, C)),
                _full((1, C)),
            ],
            out_specs=pl.BlockSpec((BR_L, C), lambda i: (i, 0)),
            out_shape=jax.ShapeDtypeStruct((N, C), f32),
        )(h, agg2, cnt2, wl, wr, bl.reshape(1, C))

    ge = pl.pallas_call(
        _gmax_body,
        grid=(N // BR_G,),
        in_specs=[
            pl.BlockSpec((BR_G, C), lambda i: (i, 0)),
            pl.BlockSpec((BR_G, 1), lambda i: (i, 0)),
        ],
        out_specs=_full((G, C)),
        out_shape=jax.ShapeDtypeStruct((G, C), f32),
    )(h, batch_f)

    hg = pl.pallas_call(
        _hg_body,
        grid=(N // BR_H,),
        in_specs=[
            pl.BlockSpec((BR_H, C), lambda i: (i, 0)),
            pl.BlockSpec((BR_H, 1), lambda i: (i, 0)),
            _full((G, C)),
        ],
        out_specs=pl.BlockSpec((BR_H, 2 * C), lambda i: (i, 0)),
        out_shape=jax.ShapeDtypeStruct((N, 2 * C), f32),
    )(h, batch_f, ge)

    sg, dg = _sc_pairs(hg, h, si, di)

    xy, pxy, pyx = pl.pallas_call(
        _head_body,
        grid=(P // BR_P,),
        in_specs=[
            pl.BlockSpec((BR_P, 2 * C), lambda i: (i, 0)),
            pl.BlockSpec((BR_P, C), lambda i: (i, 0)),
            _full((3 * C, 2 * C)),
            _full((1, 2 * C)),
            _full((2 * C, 2 * C)),
            _full((1, 2 * C)),
            _full((2 * C, 2 * C)),
            _full((1, 2 * C)),
            _full((2 * C, 1)),
            _full((1, 1)),
        ],
        out_specs=[
            pl.BlockSpec((BR_P, 3 * C), lambda i: (i, 0)),
            pl.BlockSpec((BR_P, 1), lambda i: (i, 0)),
            pl.BlockSpec((BR_P, 1), lambda i: (i, 0)),
        ],
        out_shape=[
            jax.ShapeDtypeStruct((P, 3 * C), f32),
            jax.ShapeDtypeStruct((P, 1), f32),
            jax.ShapeDtypeStruct((P, 1), f32),
        ],
    )(sg, dg, W1, b1.reshape(1, 2 * C), W2, b2.reshape(1, 2 * C),
      W3, b3.reshape(1, 2 * C), W4, b4.reshape(1, 1))

    return (h, (pxy, pyx), xy)
